# baseline (speedup 1.0000x reference)
import numpy as np

import jax
import jax.numpy as jnp
from jax import lax
from jax.experimental import pallas as pl
from jax.experimental.pallas import tpu as pltpu
from jax.experimental.pallas import tpu_sc as plsc

NB = 1_600_000
NA = 100_000
D = 32
NC = 2
NS = 16
NW = NC * NS
GR = NB // 4

GCHUNK = 2000
BONDS_PER_W = NB // NW
GCHUNKS = BONDS_PER_W // GCHUNK
WPG = GR // BONDS_PER_W


def _gather_body(atoms_hbm, idx1_hbm, idx2_hbm, a1_hbm, a2_hbm,
                 idx_v, rows_v, sem):
    c = lax.axis_index("c")
    s = lax.axis_index("s")
    wid = s * NC + c
    base = wid * BONDS_PER_W
    g = wid // WPG
    rbase = base - g * GR

    def chunk(j, carry):
        off = base + j * GCHUNK
        roff = rbase + j * GCHUNK
        pltpu.sync_copy(idx1_hbm.at[pl.ds(off, GCHUNK)], idx_v)
        pltpu.async_copy(atoms_hbm.at[idx_v], rows_v, sem).wait()
        pltpu.sync_copy(rows_v,
                        a1_hbm.at[pl.ds(roff, GCHUNK), pl.ds(g * D, D)])
        pltpu.sync_copy(idx2_hbm.at[pl.ds(off, GCHUNK)], idx_v)
        pltpu.async_copy(atoms_hbm.at[idx_v], rows_v, sem).wait()
        pltpu.sync_copy(rows_v,
                        a2_hbm.at[pl.ds(roff, GCHUNK), pl.ds(g * D, D)])
        return carry

    lax.fori_loop(0, GCHUNKS, chunk, 0)


def _sc_gather(atoms, idx1, idx2):
    mesh = plsc.VectorSubcoreMesh(core_axis_name="c", subcore_axis_name="s")
    f = pl.kernel(
        _gather_body,
        out_type=[jax.ShapeDtypeStruct((GR, 128), jnp.float32),
                  jax.ShapeDtypeStruct((GR, 128), jnp.float32)],
        mesh=mesh,
        compiler_params=pltpu.CompilerParams(use_tc_tiling_on_sc=False),
        scratch_types=[pltpu.VMEM((GCHUNK,), jnp.int32),
                       pltpu.VMEM((GCHUNK, D), jnp.float32),
                       pltpu.SemaphoreType.DMA],
    )
    return f(atoms, idx1, idx2)


ATOMS_PER_SC = NA // NC
ACC_ROWS = 50048
STRIPE = ACC_ROWS // NS
LAST_STRIPE = ATOMS_PER_SC - (NS - 1) * STRIPE
ROWS_PER_T = GR // NS
SCH = 200
SCHUNKS = ROWS_PER_T // SCH


def _scatter_body(bnew_hbm, idx_hbm, seg_hbm, cnt_hbm,
                  idx_v, lidx_v, ones_v, rows_v, feat_acc, cnt_acc):
    c = lax.axis_index("c")
    s = lax.axis_index("s")
    lo = c * ATOMS_PER_SC

    def zrow(i, carry):
        rows_v[i, pl.ds(0, 16)] = jnp.zeros((16,), jnp.float32)
        rows_v[i, pl.ds(16, 16)] = jnp.zeros((16,), jnp.float32)
        return carry

    lax.fori_loop(0, SCH, zrow, 0)

    def zone(q, carry):
        ones_v[pl.ds(jnp.minimum(q * 16, SCH - 16), 16)] = \
            jnp.zeros((16,), jnp.float32)
        return carry

    lax.fori_loop(0, (SCH + 15) // 16, zone, 0)

    nzf = STRIPE // SCH
    zrem = STRIPE - nzf * SCH

    def zcopy(k, carry):
        pltpu.sync_copy(rows_v,
                        feat_acc.at[pl.ds(s * STRIPE + k * SCH, SCH)])
        pltpu.sync_copy(ones_v.at[pl.ds(0, SCH)],
                        cnt_acc.at[pl.ds(s * STRIPE + k * SCH, SCH)])
        return carry

    lax.fori_loop(0, nzf, zcopy, 0)
    pltpu.sync_copy(rows_v.at[pl.ds(0, zrem)],
                    feat_acc.at[pl.ds(s * STRIPE + nzf * SCH, zrem)])
    pltpu.sync_copy(ones_v.at[pl.ds(0, zrem)],
                    cnt_acc.at[pl.ds(s * STRIPE + nzf * SCH, zrem)])
    plsc.subcore_barrier()

    def fone(q, carry):
        ones_v[pl.ds(jnp.minimum(q * 16, SCH - 16), 16)] = \
            jnp.ones((16,), jnp.float32)
        return carry

    lax.fori_loop(0, (SCH + 15) // 16, fone, 0)

    rb = s * ROWS_PER_T
    for g in range(4):
        def chunk(k, carry, g=g):
            r0 = rb + k * SCH
            pltpu.sync_copy(idx_hbm.at[pl.ds(g * GR + r0, SCH)], idx_v)
            pltpu.sync_copy(bnew_hbm.at[pl.ds(r0, SCH), pl.ds(g * D, D)],
                            rows_v)

            def remap(q, carry2):
                st = jnp.minimum(q * 16, SCH - 16)
                v = idx_v[pl.ds(st, 16)]
                local = v - lo
                inr = (local >= 0) & (local < ATOMS_PER_SC)
                trash = ATOMS_PER_SC + (v & 31)
                lidx_v[pl.ds(st, 16)] = jnp.where(inr, local, trash)
                return carry2

            lax.fori_loop(0, (SCH + 15) // 16, remap, 0)
            pltpu.sync_copy(rows_v, feat_acc.at[lidx_v], add=True)
            pltpu.sync_copy(ones_v.at[pl.ds(0, SCH)],
                            cnt_acc.at[lidx_v], add=True)
            return carry

        lax.fori_loop(0, SCHUNKS, chunk, 0)
    plsc.subcore_barrier()

    out_off = lo + s * STRIPE

    @pl.when(s < NS - 1)
    def _():
        pltpu.sync_copy(feat_acc.at[pl.ds(s * STRIPE, STRIPE)],
                        seg_hbm.at[pl.ds(out_off, STRIPE)])
        pltpu.sync_copy(cnt_acc.at[pl.ds(s * STRIPE, STRIPE)],
                        cnt_hbm.at[pl.ds(out_off, STRIPE)])

    @pl.when(s == NS - 1)
    def _():
        pltpu.sync_copy(feat_acc.at[pl.ds(s * STRIPE, LAST_STRIPE)],
                        seg_hbm.at[pl.ds(out_off, LAST_STRIPE)])
        pltpu.sync_copy(cnt_acc.at[pl.ds(s * STRIPE, LAST_STRIPE)],
                        cnt_hbm.at[pl.ds(out_off, LAST_STRIPE)])


def _sc_scatter(bnp, idx1):
    mesh = plsc.VectorSubcoreMesh(core_axis_name="c", subcore_axis_name="s")
    f = pl.kernel(
        _scatter_body,
        out_type=[jax.ShapeDtypeStruct((NA, D), jnp.float32),
                  jax.ShapeDtypeStruct((NA,), jnp.float32)],
        mesh=mesh,
        compiler_params=pltpu.CompilerParams(use_tc_tiling_on_sc=False),
        scratch_types=[pltpu.VMEM((SCH,), jnp.int32),
                       pltpu.VMEM((SCH,), jnp.int32),
                       pltpu.VMEM((SCH,), jnp.float32),
                       pltpu.VMEM((SCH, D), jnp.float32),
                       pltpu.VMEM_SHARED((ACC_ROWS, D), jnp.float32),
                       pltpu.VMEM_SHARED((ACC_ROWS,), jnp.float32)],
    )
    return f(bnp, idx1)


def _softplus(x):
    return jnp.maximum(x, 0.0) + jnp.log(1.0 + jnp.exp(-jnp.abs(x)))


EBLK = 3200
EGRID = GR // EBLK

_SRC = np.empty((128,), np.int32)
for _i in range(128):
    _SRC[_i] = 32 * (_i % 4) + (_i // 4)


def _edge_body(a1p, a2p, bd128, st, W1a, W1b, W1cP, W1d, b1, b2p, b3p,
               W2bd, W3bd, P, out, out_t, acc):
    i = pl.program_id(0)
    c0 = jnp.dot(st[...], W1d[...], preferred_element_type=jnp.float32) \
        + b1[...]
    c0p = jnp.concatenate([c0, c0, c0, c0], axis=1)
    bdT = jnp.transpose(bd128[...])
    h = (jnp.dot(a1p[...], W1a[...], preferred_element_type=jnp.float32)
         + jnp.dot(a2p[...], W1b[...], preferred_element_type=jnp.float32)
         + jnp.dot(bdT, W1cP[...], preferred_element_type=jnp.float32)
         + c0p)
    h = _softplus(h)
    h = _softplus(jnp.dot(h, W2bd[...], preferred_element_type=jnp.float32)
                  + b2p[...])
    h = _softplus(jnp.dot(h, W3bd[...], preferred_element_type=jnp.float32)
                  + b3p[...])
    out[...] = h
    hp = jnp.dot(h, P[...], preferred_element_type=jnp.float32)
    out_t[...] = jnp.transpose(hp)

    @pl.when(i == 0)
    def _():
        acc[...] = jnp.zeros_like(acc)

    acc[...] += jnp.sum(h.reshape(8, EBLK // 8, 128), axis=1)


def _edge_mlp(a1p, a2p, bd128, state, W1a, W1b, W1cP, W1d, b1, b2p, b3p,
              W2bd, W3bd, P):
    full = lambda shape: pl.BlockSpec(shape, lambda i: (0, 0))
    return pl.pallas_call(
        _edge_body,
        grid=(EGRID,),
        in_specs=[
            pl.BlockSpec((EBLK, 128), lambda i: (i, 0)),
            pl.BlockSpec((EBLK, 128), lambda i: (i, 0)),
            pl.BlockSpec((128, EBLK), lambda i: (0, i)),
            full((1, D)),
            full((128, 256)), full((128, 256)), full((128, 256)),
            full((32, 64)), full((1, 64)), full((1, 256)), full((1, 128)),
            full((256, 256)), full((256, 128)), full((128, 128)),
        ],
        out_specs=[
            pl.BlockSpec((EBLK, 128), lambda i: (i, 0)),
            pl.BlockSpec((128, EBLK), lambda i: (0, i)),
            pl.BlockSpec((8, 128), lambda i: (0, 0)),
        ],
        out_shape=[jax.ShapeDtypeStruct((GR, 128), jnp.float32),
                   jax.ShapeDtypeStruct((128, GR), jnp.float32),
                   jax.ShapeDtypeStruct((8, 128), jnp.float32)],
    )(a1p, a2p, bd128, state, W1a, W1b, W1cP, W1d, b1, b2p, b3p,
      W2bd, W3bd, P)


ABLK = 1000
AGRID = NA // ABLK


def _atom_body(seg, cnt, at, st, W1, b1, W2, b2, W3, b3, out, acc):
    i = pl.program_id(0)
    b2a = seg[...] / cnt[...]
    x = jnp.concatenate([b2a, at[...]], axis=1)
    c0 = jnp.dot(st[...], W1[64:96, :],
                 preferred_element_type=jnp.float32) + b1[...]
    h = jnp.dot(x, W1[0:64, :], preferred_element_type=jnp.float32) + c0
    h = _softplus(h)
    h = _softplus(jnp.dot(h, W2[...], preferred_element_type=jnp.float32)
                  + b2[...])
    h = _softplus(jnp.dot(h, W3[...], preferred_element_type=jnp.float32)
                  + b3[...])
    out[...] = h

    @pl.when(i == 0)
    def _():
        acc[...] = jnp.zeros_like(acc)

    acc[...] += jnp.sum(h.reshape(8, ABLK // 8, D), axis=1)


def _atom_mlp(seg, cnt, atoms, state, W1, b1, W2, b2, W3, b3):
    full = lambda shape: pl.BlockSpec(shape, lambda i: (0, 0))
    return pl.pallas_call(
        _atom_body,
        grid=(AGRID,),
        in_specs=[
            pl.BlockSpec((ABLK, D), lambda i: (i, 0)),
            pl.BlockSpec((ABLK, 1), lambda i: (i, 0)),
            pl.BlockSpec((ABLK, D), lambda i: (i, 0)),
            full((1, D)),
            full((96, 64)), full((1, 64)),
            full((64, 64)), full((1, 64)),
            full((64, 32)), full((1, 32)),
        ],
        out_specs=[
            pl.BlockSpec((ABLK, D), lambda i: (i, 0)),
            pl.BlockSpec((8, D), lambda i: (0, 0)),
        ],
        out_shape=[jax.ShapeDtypeStruct((NA, D), jnp.float32),
                   jax.ShapeDtypeStruct((8, D), jnp.float32)],
    )(seg, cnt, atoms, state, W1, b1, W2, b2, W3, b3)


def _state_body(bacc, aacc, st, W1, b1, W2, b2, W3, b3, out):
    bp = bacc[...]
    bsum = (bp[:, 0:32] + bp[:, 32:64] + bp[:, 64:96] + bp[:, 96:128])
    b2s = jnp.sum(bsum, axis=0, keepdims=True) / NB
    a2s = jnp.sum(aacc[...], axis=0, keepdims=True) / NA
    c0 = jnp.dot(st[...], W1[64:96, :],
                 preferred_element_type=jnp.float32) + b1[...]
    h = (jnp.dot(b2s, W1[0:32, :], preferred_element_type=jnp.float32)
         + jnp.dot(a2s, W1[32:64, :], preferred_element_type=jnp.float32)
         + c0)
    h = _softplus(h)
    h = _softplus(jnp.dot(h, W2[...], preferred_element_type=jnp.float32)
                  + b2[...])
    h = _softplus(jnp.dot(h, W3[...], preferred_element_type=jnp.float32)
                  + b3[...])
    out[...] = h


def _state_mlp(bacc, aacc, state, W1, b1, W2, b2, W3, b3):
    return pl.pallas_call(
        _state_body,
        out_shape=jax.ShapeDtypeStruct((1, D), jnp.float32),
    )(bacc, aacc, state, W1, b1, W2, b2, W3, b3)


def kernel(bonds, bond_atom_1, bond_atom_2, atoms, state,
           e_W1, e_b1, e_W2, e_b2, e_W3, e_b3,
           v_W1, v_b1, v_W2, v_b2, v_W3, v_b3,
           u_W1, u_b1, u_W2, u_b2, u_W3, u_b3):
    a1p, a2p = _sc_gather(atoms, bond_atom_1, bond_atom_2)
    eye4 = jnp.eye(4, dtype=jnp.float32)
    src = jnp.asarray(_SRC)
    bnp, bnt, bacc = _edge_mlp(
        a1p, a2p, bonds.T.reshape(128, GR), state,
        jnp.kron(eye4, e_W1[0:32, :]), jnp.kron(eye4, e_W1[32:64, :]),
        jnp.kron(eye4, e_W1[64:96, :])[src, :],
        e_W1[96:128, :], e_b1.reshape(1, 64),
        jnp.tile(e_b2, 4).reshape(1, 256), jnp.tile(e_b3, 4).reshape(1, 128),
        jnp.kron(eye4, e_W2), jnp.kron(eye4, e_W3),
        jnp.transpose(jnp.eye(128, dtype=jnp.float32)[src, :]))
    bonds_new = bnt.reshape(D, NB).T
    seg, cnt = _sc_scatter(bnp, bond_atom_1)
    atoms_new, aacc = _atom_mlp(
        seg, cnt.reshape(NA, 1), atoms, state,
        v_W1, v_b1.reshape(1, 64), v_W2, v_b2.reshape(1, 64),
        v_W3, v_b3.reshape(1, 32))
    state_new = _state_mlp(
        bacc, aacc, state,
        u_W1, u_b1.reshape(1, 64), u_W2, u_b2.reshape(1, 64),
        u_W3, u_b3.reshape(1, 32))
    return (bonds_new, atoms_new, state_new)


# speedup vs baseline: 3.2430x; 3.2430x over previous
import jax
import jax.numpy as jnp
from jax import lax
from jax.experimental import pallas as pl
from jax.experimental.pallas import tpu as pltpu
from jax.experimental.pallas import tpu_sc as plsc

NB = 1_600_000
NA = 100_000
D = 32
NC = 2
NS = 16
NW = NC * NS
EROWS = NB // 4

GCHUNK = 2000
BONDS_PER_W = NB // NW
GCHUNKS = BONDS_PER_W // GCHUNK


def _gather_body(atoms_hbm, idx1_hbm, idx2_hbm, a1_hbm, a2_hbm,
                 idx_v, rows_v, sem):
    c = lax.axis_index("c")
    s = lax.axis_index("s")
    wid = s * NC + c
    base = wid * BONDS_PER_W

    def chunk(j, carry):
        off = base + j * GCHUNK
        pltpu.sync_copy(idx1_hbm.at[pl.ds(off, GCHUNK)], idx_v)
        pltpu.async_copy(atoms_hbm.at[idx_v], rows_v, sem).wait()
        pltpu.sync_copy(rows_v, a1_hbm.at[pl.ds(off, GCHUNK)])
        pltpu.sync_copy(idx2_hbm.at[pl.ds(off, GCHUNK)], idx_v)
        pltpu.async_copy(atoms_hbm.at[idx_v], rows_v, sem).wait()
        pltpu.sync_copy(rows_v, a2_hbm.at[pl.ds(off, GCHUNK)])
        return carry

    lax.fori_loop(0, GCHUNKS, chunk, 0)


def _sc_gather(atoms_bf, idx1, idx2):
    mesh = plsc.VectorSubcoreMesh(core_axis_name="c", subcore_axis_name="s")
    f = pl.kernel(
        _gather_body,
        out_type=[jax.ShapeDtypeStruct((NB, D), jnp.bfloat16),
                  jax.ShapeDtypeStruct((NB, D), jnp.bfloat16)],
        mesh=mesh,
        compiler_params=pltpu.CompilerParams(use_tc_tiling_on_sc=False),
        scratch_types=[pltpu.VMEM((GCHUNK,), jnp.int32),
                       pltpu.VMEM((GCHUNK, D), jnp.bfloat16),
                       pltpu.SemaphoreType.DMA],
    )
    return f(atoms_bf, idx1, idx2)


ATOMS_PER_SC = NA // NC
ACC_ROWS = 50176
STRIPE = ACC_ROWS // NS
LAST_STRIPE = ATOMS_PER_SC - (NS - 1) * STRIPE
SCHUNK = 400
BONDS_PER_T = NB // NS
SCHUNKS = BONDS_PER_T // SCHUNK
VGRP = SCHUNK // 16


def _scatter_body(bnew_hbm, idx_hbm, seg_hbm, cnt_hbm,
                  idx_v, lidx_v, rows_v, ones_v, feat_acc, cnt_acc):
    c = lax.axis_index("c")
    s = lax.axis_index("s")
    lo = c * ATOMS_PER_SC

    def zrow(i, carry):
        rows_v[i, pl.ds(0, 16)] = jnp.zeros((16,), jnp.float32)
        rows_v[i, pl.ds(16, 16)] = jnp.zeros((16,), jnp.float32)
        return carry

    lax.fori_loop(0, SCHUNK, zrow, 0)

    def zone(q, carry):
        ones_v[pl.ds(q * 16, 16)] = jnp.zeros((16,), jnp.float32)
        return carry

    lax.fori_loop(0, VGRP, zone, 0)

    nfull = STRIPE // SCHUNK
    rem = STRIPE - nfull * SCHUNK

    def zcopy(k, carry):
        pltpu.sync_copy(rows_v,
                        feat_acc.at[pl.ds(s * STRIPE + k * SCHUNK, SCHUNK)])
        pltpu.sync_copy(ones_v,
                        cnt_acc.at[pl.ds(s * STRIPE + k * SCHUNK, SCHUNK)])
        return carry

    lax.fori_loop(0, nfull, zcopy, 0)
    pltpu.sync_copy(rows_v.at[pl.ds(0, rem)],
                    feat_acc.at[pl.ds(s * STRIPE + nfull * SCHUNK, rem)])
    pltpu.sync_copy(ones_v.at[pl.ds(0, rem)],
                    cnt_acc.at[pl.ds(s * STRIPE + nfull * SCHUNK, rem)])
    plsc.subcore_barrier()

    def fone(q, carry):
        ones_v[pl.ds(q * 16, 16)] = jnp.ones((16,), jnp.float32)
        return carry

    lax.fori_loop(0, VGRP, fone, 0)

    base = s * BONDS_PER_T

    def chunk(j, carry):
        off = base + j * SCHUNK
        pltpu.sync_copy(idx_hbm.at[pl.ds(off, SCHUNK)], idx_v)
        pltpu.sync_copy(bnew_hbm.at[pl.ds(off, SCHUNK)], rows_v)

        def remap(g, carry2):
            v = idx_v[pl.ds(g * 16, 16)]
            local = v - lo
            inr = (local >= 0) & (local < ATOMS_PER_SC)
            trash = ATOMS_PER_SC + (v & 127)
            lidx_v[pl.ds(g * 16, 16)] = jnp.where(inr, local, trash)
            return carry2

        lax.fori_loop(0, VGRP, remap, 0)
        pltpu.sync_copy(rows_v, feat_acc.at[lidx_v], add=True)
        pltpu.sync_copy(ones_v, cnt_acc.at[lidx_v], add=True)
        return carry

    lax.fori_loop(0, SCHUNKS, chunk, 0)
    plsc.subcore_barrier()

    out_off = lo + s * STRIPE

    @pl.when(s < NS - 1)
    def _():
        pltpu.sync_copy(feat_acc.at[pl.ds(s * STRIPE, STRIPE)],
                        seg_hbm.at[pl.ds(out_off, STRIPE)])
        pltpu.sync_copy(cnt_acc.at[pl.ds(s * STRIPE, STRIPE)],
                        cnt_hbm.at[pl.ds(out_off, STRIPE)])

    @pl.when(s == NS - 1)
    def _():
        pltpu.sync_copy(feat_acc.at[pl.ds(s * STRIPE, LAST_STRIPE)],
                        seg_hbm.at[pl.ds(out_off, LAST_STRIPE)])
        pltpu.sync_copy(cnt_acc.at[pl.ds(s * STRIPE, LAST_STRIPE)],
                        cnt_hbm.at[pl.ds(out_off, LAST_STRIPE)])


def _sc_scatter(bonds_new, idx1):
    mesh = plsc.VectorSubcoreMesh(core_axis_name="c", subcore_axis_name="s")
    f = pl.kernel(
        _scatter_body,
        out_type=[jax.ShapeDtypeStruct((NA, D), jnp.float32),
                  jax.ShapeDtypeStruct((NA,), jnp.float32)],
        mesh=mesh,
        compiler_params=pltpu.CompilerParams(use_tc_tiling_on_sc=False),
        scratch_types=[pltpu.VMEM((SCHUNK,), jnp.int32),
                       pltpu.VMEM((SCHUNK,), jnp.int32),
                       pltpu.VMEM((SCHUNK, D), jnp.float32),
                       pltpu.VMEM((SCHUNK,), jnp.float32),
                       pltpu.VMEM_SHARED((ACC_ROWS, D), jnp.float32),
                       pltpu.VMEM_SHARED((ACC_ROWS,), jnp.float32)],
    )
    return f(bonds_new, idx1)


def _softplus(x):
    return jnp.maximum(x, 0.0) + jnp.log(1.0 + jnp.exp(-jnp.abs(x)))


EBLK = 1600
EGRID = EROWS // EBLK


def _edge_body(a1p, a2p, bdp, st, W1a, W1b, W1c, W1d, b1, b2p, b3p,
               W2bd, W3bd, out, acc):
    i = pl.program_id(0)
    c0 = jnp.dot(st[...], W1d[...], preferred_element_type=jnp.float32) \
        + b1[...]
    c0p = jnp.concatenate([c0, c0, c0, c0], axis=1)
    h = (jnp.dot(a1p[...], W1a[...], preferred_element_type=jnp.float32)
         + jnp.dot(a2p[...], W1b[...], preferred_element_type=jnp.float32)
         + jnp.dot(bdp[...], W1c[...], preferred_element_type=jnp.float32)
         + c0p)
    h = _softplus(h)
    h = _softplus(jnp.dot(h, W2bd[...], preferred_element_type=jnp.float32)
                  + b2p[...])
    h = _softplus(jnp.dot(h, W3bd[...], preferred_element_type=jnp.float32)
                  + b3p[...])
    out[...] = h

    @pl.when(i == 0)
    def _():
        acc[...] = jnp.zeros_like(acc)

    acc[...] += jnp.sum(h.reshape(8, EBLK // 8, 128), axis=1)


def _edge_mlp(a1p, a2p, bdp, state, W1a, W1b, W1c, W1d, b1, b2p, b3p,
              W2bd, W3bd):
    full = lambda shape: pl.BlockSpec(shape, lambda i: (0, 0))
    return pl.pallas_call(
        _edge_body,
        grid=(EGRID,),
        in_specs=[
            pl.BlockSpec((EBLK, 128), lambda i: (i, 0)),
            pl.BlockSpec((EBLK, 128), lambda i: (i, 0)),
            pl.BlockSpec((EBLK, 128), lambda i: (i, 0)),
            full((1, D)),
            full((128, 256)), full((128, 256)), full((128, 256)),
            full((32, 64)), full((1, 64)), full((1, 256)), full((1, 128)),
            full((256, 256)), full((256, 128)),
        ],
        out_specs=[
            pl.BlockSpec((EBLK, 128), lambda i: (i, 0)),
            pl.BlockSpec((8, 128), lambda i: (0, 0)),
        ],
        out_shape=[jax.ShapeDtypeStruct((EROWS, 128), jnp.float32),
                   jax.ShapeDtypeStruct((8, 128), jnp.float32)],
    )(a1p, a2p, bdp, state, W1a, W1b, W1c, W1d, b1, b2p, b3p, W2bd, W3bd)


ABLK = 1000
AGRID = NA // ABLK


def _atom_body(seg, cnt, at, st, W1, b1, W2, b2, W3, b3, out, acc):
    i = pl.program_id(0)
    b2a = seg[...] / cnt[...]
    x = jnp.concatenate([b2a, at[...]], axis=1)
    c0 = jnp.dot(st[...], W1[64:96, :],
                 preferred_element_type=jnp.float32) + b1[...]
    h = jnp.dot(x, W1[0:64, :], preferred_element_type=jnp.float32) + c0
    h = _softplus(h)
    h = _softplus(jnp.dot(h, W2[...], preferred_element_type=jnp.float32)
                  + b2[...])
    h = _softplus(jnp.dot(h, W3[...], preferred_element_type=jnp.float32)
                  + b3[...])
    out[...] = h

    @pl.when(i == 0)
    def _():
        acc[...] = jnp.zeros_like(acc)

    acc[...] += jnp.sum(h.reshape(8, ABLK // 8, D), axis=1)


def _atom_mlp(seg, cnt, atoms, state, W1, b1, W2, b2, W3, b3):
    full = lambda shape: pl.BlockSpec(shape, lambda i: (0, 0))
    return pl.pallas_call(
        _atom_body,
        grid=(AGRID,),
        in_specs=[
            pl.BlockSpec((ABLK, D), lambda i: (i, 0)),
            pl.BlockSpec((ABLK, 1), lambda i: (i, 0)),
            pl.BlockSpec((ABLK, D), lambda i: (i, 0)),
            full((1, D)),
            full((96, 64)), full((1, 64)),
            full((64, 64)), full((1, 64)),
            full((64, 32)), full((1, 32)),
        ],
        out_specs=[
            pl.BlockSpec((ABLK, D), lambda i: (i, 0)),
            pl.BlockSpec((8, D), lambda i: (0, 0)),
        ],
        out_shape=[jax.ShapeDtypeStruct((NA, D), jnp.float32),
                   jax.ShapeDtypeStruct((8, D), jnp.float32)],
    )(seg, cnt, atoms, state, W1, b1, W2, b2, W3, b3)


def _state_body(bacc, aacc, st, W1, b1, W2, b2, W3, b3, out):
    bp = bacc[...]
    bsum = (bp[:, 0:32] + bp[:, 32:64] + bp[:, 64:96] + bp[:, 96:128])
    b2s = jnp.sum(bsum, axis=0, keepdims=True) / NB
    a2s = jnp.sum(aacc[...], axis=0, keepdims=True) / NA
    c0 = jnp.dot(st[...], W1[64:96, :],
                 preferred_element_type=jnp.float32) + b1[...]
    h = (jnp.dot(b2s, W1[0:32, :], preferred_element_type=jnp.float32)
         + jnp.dot(a2s, W1[32:64, :], preferred_element_type=jnp.float32)
         + c0)
    h = _softplus(h)
    h = _softplus(jnp.dot(h, W2[...], preferred_element_type=jnp.float32)
                  + b2[...])
    h = _softplus(jnp.dot(h, W3[...], preferred_element_type=jnp.float32)
                  + b3[...])
    out[...] = h


def _state_mlp(bacc, aacc, state, W1, b1, W2, b2, W3, b3):
    return pl.pallas_call(
        _state_body,
        out_shape=jax.ShapeDtypeStruct((1, D), jnp.float32),
    )(bacc, aacc, state, W1, b1, W2, b2, W3, b3)


def kernel(bonds, bond_atom_1, bond_atom_2, atoms, state,
           e_W1, e_b1, e_W2, e_b2, e_W3, e_b3,
           v_W1, v_b1, v_W2, v_b2, v_W3, v_b3,
           u_W1, u_b1, u_W2, u_b2, u_W3, u_b3):
    a1, a2 = _sc_gather(atoms.astype(jnp.bfloat16), bond_atom_1, bond_atom_2)
    eye4 = jnp.eye(4, dtype=jnp.float32)
    bf = jnp.bfloat16
    bnp, bacc = _edge_mlp(
        a1.reshape(EROWS, 128), a2.reshape(EROWS, 128),
        bonds.reshape(EROWS, 128), state,
        jnp.kron(eye4, e_W1[0:32, :]).astype(bf),
        jnp.kron(eye4, e_W1[32:64, :]).astype(bf),
        jnp.kron(eye4, e_W1[64:96, :]),
        e_W1[96:128, :], e_b1.reshape(1, 64),
        jnp.tile(e_b2, 4).reshape(1, 256), jnp.tile(e_b3, 4).reshape(1, 128),
        jnp.kron(eye4, e_W2), jnp.kron(eye4, e_W3))
    bonds_new = bnp.reshape(NB, D)
    seg, cnt = _sc_scatter(bonds_new, bond_atom_1)
    atoms_new, aacc = _atom_mlp(
        seg, cnt.reshape(NA, 1), atoms, state,
        v_W1, v_b1.reshape(1, 64), v_W2, v_b2.reshape(1, 64),
        v_W3, v_b3.reshape(1, 32))
    state_new = _state_mlp(
        bacc, aacc, state,
        u_W1, u_b1.reshape(1, 64), u_W2, u_b2.reshape(1, 64),
        u_W3, u_b3.reshape(1, 32))
    return (bonds_new, atoms_new, state_new)


# speedup vs baseline: 4.1186x; 1.2700x over previous
import jax
import jax.numpy as jnp
from jax import lax
from jax.experimental import pallas as pl
from jax.experimental.pallas import tpu as pltpu
from jax.experimental.pallas import tpu_sc as plsc

NB = 1_600_000
NA = 100_000
D = 32
NC = 2
NS = 16
NW = NC * NS
EROWS = NB // 4

GCHUNK = 2000
BONDS_PER_W = NB // NW
GCHUNKS = BONDS_PER_W // GCHUNK


def _gather_body(atoms_hbm, idx1_hbm, idx2_hbm, a1_hbm, a2_hbm,
                 idx_v, rows_v, sem):
    c = lax.axis_index("c")
    s = lax.axis_index("s")
    wid = s * NC + c
    base = wid * BONDS_PER_W

    def chunk(j, carry):
        off = base + j * GCHUNK
        pltpu.sync_copy(idx1_hbm.at[pl.ds(off, GCHUNK)], idx_v)
        pltpu.async_copy(atoms_hbm.at[idx_v], rows_v, sem).wait()
        pltpu.sync_copy(rows_v, a1_hbm.at[pl.ds(off, GCHUNK)])
        pltpu.sync_copy(idx2_hbm.at[pl.ds(off, GCHUNK)], idx_v)
        pltpu.async_copy(atoms_hbm.at[idx_v], rows_v, sem).wait()
        pltpu.sync_copy(rows_v, a2_hbm.at[pl.ds(off, GCHUNK)])
        return carry

    lax.fori_loop(0, GCHUNKS, chunk, 0)


def _sc_gather(atoms, idx1, idx2):
    mesh = plsc.VectorSubcoreMesh(core_axis_name="c", subcore_axis_name="s")
    f = pl.kernel(
        _gather_body,
        out_type=[jax.ShapeDtypeStruct((NB, D), jnp.float32),
                  jax.ShapeDtypeStruct((NB, D), jnp.float32)],
        mesh=mesh,
        compiler_params=pltpu.CompilerParams(use_tc_tiling_on_sc=False),
        scratch_types=[pltpu.VMEM((GCHUNK,), jnp.int32),
                       pltpu.VMEM((GCHUNK, D), jnp.float32),
                       pltpu.SemaphoreType.DMA],
    )
    return f(atoms, idx1, idx2)


ATOMS_PER_SC = NA // NC
ACC_ROWS = 50176
STRIPE = ACC_ROWS // NS
LAST_STRIPE = ATOMS_PER_SC - (NS - 1) * STRIPE
SCHUNK = 400
BONDS_PER_T = NB // NS
SCHUNKS = BONDS_PER_T // SCHUNK
VGRP = SCHUNK // 16


def _scatter_body(bnew_hbm, idx_hbm, seg_hbm, cnt_hbm,
                  idx_v, lidx_v, rows_v, ones_v, feat_acc, cnt_acc):
    c = lax.axis_index("c")
    s = lax.axis_index("s")
    lo = c * ATOMS_PER_SC

    def zrow(i, carry):
        rows_v[i, pl.ds(0, 16)] = jnp.zeros((16,), jnp.float32)
        rows_v[i, pl.ds(16, 16)] = jnp.zeros((16,), jnp.float32)
        return carry

    lax.fori_loop(0, SCHUNK, zrow, 0)

    def zone(q, carry):
        ones_v[pl.ds(q * 16, 16)] = jnp.zeros((16,), jnp.float32)
        return carry

    lax.fori_loop(0, VGRP, zone, 0)

    nfull = STRIPE // SCHUNK
    rem = STRIPE - nfull * SCHUNK

    def zcopy(k, carry):
        pltpu.sync_copy(rows_v,
                        feat_acc.at[pl.ds(s * STRIPE + k * SCHUNK, SCHUNK)])
        pltpu.sync_copy(ones_v,
                        cnt_acc.at[pl.ds(s * STRIPE + k * SCHUNK, SCHUNK)])
        return carry

    lax.fori_loop(0, nfull, zcopy, 0)
    pltpu.sync_copy(rows_v.at[pl.ds(0, rem)],
                    feat_acc.at[pl.ds(s * STRIPE + nfull * SCHUNK, rem)])
    pltpu.sync_copy(ones_v.at[pl.ds(0, rem)],
                    cnt_acc.at[pl.ds(s * STRIPE + nfull * SCHUNK, rem)])
    plsc.subcore_barrier()

    def fone(q, carry):
        ones_v[pl.ds(q * 16, 16)] = jnp.ones((16,), jnp.float32)
        return carry

    lax.fori_loop(0, VGRP, fone, 0)

    base = s * BONDS_PER_T

    def chunk(j, carry):
        off = base + j * SCHUNK
        pltpu.sync_copy(idx_hbm.at[pl.ds(off, SCHUNK)], idx_v)
        pltpu.sync_copy(bnew_hbm.at[pl.ds(off, SCHUNK)], rows_v)

        def remap(g, carry2):
            v = idx_v[pl.ds(g * 16, 16)]
            local = v - lo
            inr = (local >= 0) & (local < ATOMS_PER_SC)
            trash = ATOMS_PER_SC + (v & 127)
            lidx_v[pl.ds(g * 16, 16)] = jnp.where(inr, local, trash)
            return carry2

        lax.fori_loop(0, VGRP, remap, 0)
        pltpu.sync_copy(rows_v, feat_acc.at[lidx_v], add=True)
        pltpu.sync_copy(ones_v, cnt_acc.at[lidx_v], add=True)
        return carry

    lax.fori_loop(0, SCHUNKS, chunk, 0)
    plsc.subcore_barrier()

    out_off = lo + s * STRIPE

    @pl.when(s < NS - 1)
    def _():
        pltpu.sync_copy(feat_acc.at[pl.ds(s * STRIPE, STRIPE)],
                        seg_hbm.at[pl.ds(out_off, STRIPE)])
        pltpu.sync_copy(cnt_acc.at[pl.ds(s * STRIPE, STRIPE)],
                        cnt_hbm.at[pl.ds(out_off, STRIPE)])

    @pl.when(s == NS - 1)
    def _():
        pltpu.sync_copy(feat_acc.at[pl.ds(s * STRIPE, LAST_STRIPE)],
                        seg_hbm.at[pl.ds(out_off, LAST_STRIPE)])
        pltpu.sync_copy(cnt_acc.at[pl.ds(s * STRIPE, LAST_STRIPE)],
                        cnt_hbm.at[pl.ds(out_off, LAST_STRIPE)])


def _sc_scatter(bonds_new, idx1):
    mesh = plsc.VectorSubcoreMesh(core_axis_name="c", subcore_axis_name="s")
    f = pl.kernel(
        _scatter_body,
        out_type=[jax.ShapeDtypeStruct((NA, D), jnp.float32),
                  jax.ShapeDtypeStruct((NA,), jnp.float32)],
        mesh=mesh,
        compiler_params=pltpu.CompilerParams(use_tc_tiling_on_sc=False),
        scratch_types=[pltpu.VMEM((SCHUNK,), jnp.int32),
                       pltpu.VMEM((SCHUNK,), jnp.int32),
                       pltpu.VMEM((SCHUNK, D), jnp.float32),
                       pltpu.VMEM((SCHUNK,), jnp.float32),
                       pltpu.VMEM_SHARED((ACC_ROWS, D), jnp.float32),
                       pltpu.VMEM_SHARED((ACC_ROWS,), jnp.float32)],
    )
    return f(bonds_new, idx1)


def _softplus(x):
    return jnp.maximum(x, 0.0) + jnp.log(1.0 + jnp.exp(-jnp.abs(x)))


EBLK = 1600
EGRID = EROWS // EBLK


def _edge_body(a1p, a2p, bdp, st, W1a, W1b, W1c, W1d, b1, b2p, b3p,
               W2bd, W3bd, out, acc):
    i = pl.program_id(0)
    c0 = jnp.dot(st[...], W1d[...], preferred_element_type=jnp.float32) \
        + b1[...]
    c0p = jnp.concatenate([c0, c0, c0, c0], axis=1)
    h = (jnp.dot(a1p[...], W1a[...], preferred_element_type=jnp.float32)
         + jnp.dot(a2p[...], W1b[...], preferred_element_type=jnp.float32)
         + jnp.dot(bdp[...], W1c[...], preferred_element_type=jnp.float32)
         + c0p)
    h = _softplus(h)
    h = _softplus(jnp.dot(h, W2bd[...], preferred_element_type=jnp.float32)
                  + b2p[...])
    h = _softplus(jnp.dot(h, W3bd[...], preferred_element_type=jnp.float32)
                  + b3p[...])
    out[...] = h

    @pl.when(i == 0)
    def _():
        acc[...] = jnp.zeros_like(acc)

    acc[...] += jnp.sum(h.reshape(8, EBLK // 8, 128), axis=1)


def _edge_mlp(a1p, a2p, bdp, state, W1a, W1b, W1c, W1d, b1, b2p, b3p,
              W2bd, W3bd):
    full = lambda shape: pl.BlockSpec(shape, lambda i: (0, 0))
    return pl.pallas_call(
        _edge_body,
        grid=(EGRID,),
        in_specs=[
            pl.BlockSpec((EBLK, 128), lambda i: (i, 0)),
            pl.BlockSpec((EBLK, 128), lambda i: (i, 0)),
            pl.BlockSpec((EBLK, 128), lambda i: (i, 0)),
            full((1, D)),
            full((128, 256)), full((128, 256)), full((128, 256)),
            full((32, 64)), full((1, 64)), full((1, 256)), full((1, 128)),
            full((256, 256)), full((256, 128)),
        ],
        out_specs=[
            pl.BlockSpec((EBLK, 128), lambda i: (i, 0)),
            pl.BlockSpec((8, 128), lambda i: (0, 0)),
        ],
        out_shape=[jax.ShapeDtypeStruct((EROWS, 128), jnp.float32),
                   jax.ShapeDtypeStruct((8, 128), jnp.float32)],
    )(a1p, a2p, bdp, state, W1a, W1b, W1c, W1d, b1, b2p, b3p, W2bd, W3bd)


ABLK = 1000
AGRID = NA // ABLK


def _atom_body(seg, cnt, at, st, W1, b1, W2, b2, W3, b3, out, acc):
    i = pl.program_id(0)
    b2a = seg[...] / cnt[...]
    x = jnp.concatenate([b2a, at[...]], axis=1)
    c0 = jnp.dot(st[...], W1[64:96, :],
                 preferred_element_type=jnp.float32) + b1[...]
    h = jnp.dot(x, W1[0:64, :], preferred_element_type=jnp.float32) + c0
    h = _softplus(h)
    h = _softplus(jnp.dot(h, W2[...], preferred_element_type=jnp.float32)
                  + b2[...])
    h = _softplus(jnp.dot(h, W3[...], preferred_element_type=jnp.float32)
                  + b3[...])
    out[...] = h

    @pl.when(i == 0)
    def _():
        acc[...] = jnp.zeros_like(acc)

    acc[...] += jnp.sum(h.reshape(8, ABLK // 8, D), axis=1)


def _atom_mlp(seg, cnt, atoms, state, W1, b1, W2, b2, W3, b3):
    full = lambda shape: pl.BlockSpec(shape, lambda i: (0, 0))
    return pl.pallas_call(
        _atom_body,
        grid=(AGRID,),
        in_specs=[
            pl.BlockSpec((ABLK, D), lambda i: (i, 0)),
            pl.BlockSpec((ABLK, 1), lambda i: (i, 0)),
            pl.BlockSpec((ABLK, D), lambda i: (i, 0)),
            full((1, D)),
            full((96, 64)), full((1, 64)),
            full((64, 64)), full((1, 64)),
            full((64, 32)), full((1, 32)),
        ],
        out_specs=[
            pl.BlockSpec((ABLK, D), lambda i: (i, 0)),
            pl.BlockSpec((8, D), lambda i: (0, 0)),
        ],
        out_shape=[jax.ShapeDtypeStruct((NA, D), jnp.float32),
                   jax.ShapeDtypeStruct((8, D), jnp.float32)],
    )(seg, cnt, atoms, state, W1, b1, W2, b2, W3, b3)


def _state_body(bacc, aacc, st, W1, b1, W2, b2, W3, b3, out):
    bp = bacc[...]
    bsum = (bp[:, 0:32] + bp[:, 32:64] + bp[:, 64:96] + bp[:, 96:128])
    b2s = jnp.sum(bsum, axis=0, keepdims=True) / NB
    a2s = jnp.sum(aacc[...], axis=0, keepdims=True) / NA
    c0 = jnp.dot(st[...], W1[64:96, :],
                 preferred_element_type=jnp.float32) + b1[...]
    h = (jnp.dot(b2s, W1[0:32, :], preferred_element_type=jnp.float32)
         + jnp.dot(a2s, W1[32:64, :], preferred_element_type=jnp.float32)
         + c0)
    h = _softplus(h)
    h = _softplus(jnp.dot(h, W2[...], preferred_element_type=jnp.float32)
                  + b2[...])
    h = _softplus(jnp.dot(h, W3[...], preferred_element_type=jnp.float32)
                  + b3[...])
    out[...] = h


def _state_mlp(bacc, aacc, state, W1, b1, W2, b2, W3, b3):
    return pl.pallas_call(
        _state_body,
        out_shape=jax.ShapeDtypeStruct((1, D), jnp.float32),
    )(bacc, aacc, state, W1, b1, W2, b2, W3, b3)


def kernel(bonds, bond_atom_1, bond_atom_2, atoms, state,
           e_W1, e_b1, e_W2, e_b2, e_W3, e_b3,
           v_W1, v_b1, v_W2, v_b2, v_W3, v_b3,
           u_W1, u_b1, u_W2, u_b2, u_W3, u_b3):
    a1, a2 = _sc_gather(atoms, bond_atom_1, bond_atom_2)
    eye4 = jnp.eye(4, dtype=jnp.float32)
    bnp, bacc = _edge_mlp(
        a1.reshape(EROWS, 128), a2.reshape(EROWS, 128),
        bonds.reshape(EROWS, 128), state,
        jnp.kron(eye4, e_W1[0:32, :]),
        jnp.kron(eye4, e_W1[32:64, :]),
        jnp.kron(eye4, e_W1[64:96, :]),
        e_W1[96:128, :], e_b1.reshape(1, 64),
        jnp.tile(e_b2, 4).reshape(1, 256), jnp.tile(e_b3, 4).reshape(1, 128),
        jnp.kron(eye4, e_W2), jnp.kron(eye4, e_W3))
    bonds_new = bnp.reshape(NB, D)
    seg, cnt = _sc_scatter(bonds_new, bond_atom_1)
    atoms_new, aacc = _atom_mlp(
        seg, cnt.reshape(NA, 1), atoms, state,
        v_W1, v_b1.reshape(1, 64), v_W2, v_b2.reshape(1, 64),
        v_W3, v_b3.reshape(1, 32))
    state_new = _state_mlp(
        bacc, aacc, state,
        u_W1, u_b1.reshape(1, 64), u_W2, u_b2.reshape(1, 64),
        u_W3, u_b3.reshape(1, 32))
    return (bonds_new, atoms_new, state_new)


# speedup vs baseline: 4.4219x; 1.0737x over previous
import jax
import jax.numpy as jnp
from jax import lax
from jax.experimental import pallas as pl
from jax.experimental.pallas import tpu as pltpu
from jax.experimental.pallas import tpu_sc as plsc

NB = 1_600_000
NA = 100_000
D = 32
NC = 2
NS = 16
NW = NC * NS
EROWS = NB // 4

GCHUNK = 2000
BONDS_PER_W = NB // NW
GCHUNKS = BONDS_PER_W // GCHUNK


def _gather_body(atoms_hbm, idx1_hbm, idx2_hbm, a1_hbm, a2_hbm,
                 idx_v, rows_v, sem):
    c = lax.axis_index("c")
    s = lax.axis_index("s")
    wid = s * NC + c
    base = wid * BONDS_PER_W

    def chunk(j, carry):
        off = base + j * GCHUNK
        pltpu.sync_copy(idx1_hbm.at[pl.ds(off, GCHUNK)], idx_v)
        pltpu.async_copy(atoms_hbm.at[idx_v], rows_v, sem).wait()
        pltpu.sync_copy(rows_v, a1_hbm.at[pl.ds(off, GCHUNK)])
        pltpu.sync_copy(idx2_hbm.at[pl.ds(off, GCHUNK)], idx_v)
        pltpu.async_copy(atoms_hbm.at[idx_v], rows_v, sem).wait()
        pltpu.sync_copy(rows_v, a2_hbm.at[pl.ds(off, GCHUNK)])
        return carry

    lax.fori_loop(0, GCHUNKS, chunk, 0)


def _sc_gather(atoms, idx1, idx2):
    mesh = plsc.VectorSubcoreMesh(core_axis_name="c", subcore_axis_name="s")
    f = pl.kernel(
        _gather_body,
        out_type=[jax.ShapeDtypeStruct((NB, D), jnp.float32),
                  jax.ShapeDtypeStruct((NB, D), jnp.float32)],
        mesh=mesh,
        compiler_params=pltpu.CompilerParams(use_tc_tiling_on_sc=False),
        scratch_types=[pltpu.VMEM((GCHUNK,), jnp.int32),
                       pltpu.VMEM((GCHUNK, D), jnp.float32),
                       pltpu.SemaphoreType.DMA],
    )
    return f(atoms, idx1, idx2)


ATOMS_PER_SC = NA // NC
ACC_ROWS = 50048
STRIPE = ACC_ROWS // NS
LAST_STRIPE = ATOMS_PER_SC - (NS - 1) * STRIPE
SCHUNK = 400
BONDS_PER_T = NB // NS
SCHUNKS = BONDS_PER_T // SCHUNK
VGRP = SCHUNK // 16


def _scatter_body(bnew_hbm, idx_hbm, seg_hbm, cnt_hbm,
                  idx_a, idx_b, lidx_v, rows_a, rows_b, ones_v,
                  feat_acc, cnt_acc, sia, sra, sib, srb):
    c = lax.axis_index("c")
    s = lax.axis_index("s")
    lo = c * ATOMS_PER_SC

    def zrow(i, carry):
        rows_a[i, pl.ds(0, 16)] = jnp.zeros((16,), jnp.float32)
        rows_a[i, pl.ds(16, 16)] = jnp.zeros((16,), jnp.float32)
        return carry

    lax.fori_loop(0, SCHUNK, zrow, 0)

    def zone(q, carry):
        ones_v[pl.ds(q * 16, 16)] = jnp.zeros((16,), jnp.float32)
        return carry

    lax.fori_loop(0, VGRP, zone, 0)

    nfull = STRIPE // SCHUNK
    rem = STRIPE - nfull * SCHUNK

    def zcopy(k, carry):
        pltpu.sync_copy(rows_a,
                        feat_acc.at[pl.ds(s * STRIPE + k * SCHUNK, SCHUNK)])
        pltpu.sync_copy(ones_v,
                        cnt_acc.at[pl.ds(s * STRIPE + k * SCHUNK, SCHUNK)])
        return carry

    lax.fori_loop(0, nfull, zcopy, 0)
    pltpu.sync_copy(rows_a.at[pl.ds(0, rem)],
                    feat_acc.at[pl.ds(s * STRIPE + nfull * SCHUNK, rem)])
    pltpu.sync_copy(ones_v.at[pl.ds(0, rem)],
                    cnt_acc.at[pl.ds(s * STRIPE + nfull * SCHUNK, rem)])
    plsc.subcore_barrier()

    def fone(q, carry):
        ones_v[pl.ds(q * 16, 16)] = jnp.ones((16,), jnp.float32)
        return carry

    lax.fori_loop(0, VGRP, fone, 0)

    base = s * BONDS_PER_T

    def remap(idx_ref):
        def rbody(g, carry2):
            v = idx_ref[pl.ds(g * 16, 16)]
            local = v - lo
            inr = (local >= 0) & (local < ATOMS_PER_SC)
            trash = ATOMS_PER_SC + (v & 31)
            lidx_v[pl.ds(g * 16, 16)] = jnp.where(inr, local, trash)
            return carry2

        lax.fori_loop(0, VGRP, rbody, 0)

    def scat(rows_ref):
        pltpu.sync_copy(rows_ref, feat_acc.at[lidx_v], add=True)
        pltpu.sync_copy(ones_v, cnt_acc.at[lidx_v], add=True)

    pltpu.async_copy(idx_hbm.at[pl.ds(base, SCHUNK)], idx_a, sia)
    pltpu.async_copy(bnew_hbm.at[pl.ds(base, SCHUNK)], rows_a, sra)

    def pair(j, carry):
        o1 = base + (2 * j + 1) * SCHUNK
        db_i = pltpu.async_copy(idx_hbm.at[pl.ds(o1, SCHUNK)], idx_b, sib)
        db_r = pltpu.async_copy(bnew_hbm.at[pl.ds(o1, SCHUNK)], rows_b, srb)
        pltpu.make_async_copy(idx_hbm.at[pl.ds(base, SCHUNK)],
                              idx_a, sia).wait()
        pltpu.make_async_copy(bnew_hbm.at[pl.ds(base, SCHUNK)],
                              rows_a, sra).wait()
        remap(idx_a)
        scat(rows_a)

        @pl.when(j < SCHUNKS // 2 - 1)
        def _():
            o2 = base + (2 * j + 2) * SCHUNK
            pltpu.async_copy(idx_hbm.at[pl.ds(o2, SCHUNK)], idx_a, sia)
            pltpu.async_copy(bnew_hbm.at[pl.ds(o2, SCHUNK)], rows_a, sra)

        db_i.wait()
        db_r.wait()
        remap(idx_b)
        scat(rows_b)
        return carry

    lax.fori_loop(0, SCHUNKS // 2, pair, 0)
    plsc.subcore_barrier()

    out_off = lo + s * STRIPE

    @pl.when(s < NS - 1)
    def _():
        pltpu.sync_copy(feat_acc.at[pl.ds(s * STRIPE, STRIPE)],
                        seg_hbm.at[pl.ds(out_off, STRIPE)])
        pltpu.sync_copy(cnt_acc.at[pl.ds(s * STRIPE, STRIPE)],
                        cnt_hbm.at[pl.ds(out_off, STRIPE)])

    @pl.when(s == NS - 1)
    def _():
        pltpu.sync_copy(feat_acc.at[pl.ds(s * STRIPE, LAST_STRIPE)],
                        seg_hbm.at[pl.ds(out_off, LAST_STRIPE)])
        pltpu.sync_copy(cnt_acc.at[pl.ds(s * STRIPE, LAST_STRIPE)],
                        cnt_hbm.at[pl.ds(out_off, LAST_STRIPE)])


def _sc_scatter(bonds_new, idx1):
    mesh = plsc.VectorSubcoreMesh(core_axis_name="c", subcore_axis_name="s")
    f = pl.kernel(
        _scatter_body,
        out_type=[jax.ShapeDtypeStruct((NA, D), jnp.float32),
                  jax.ShapeDtypeStruct((NA,), jnp.float32)],
        mesh=mesh,
        compiler_params=pltpu.CompilerParams(use_tc_tiling_on_sc=False),
        scratch_types=[pltpu.VMEM((SCHUNK,), jnp.int32),
                       pltpu.VMEM((SCHUNK,), jnp.int32),
                       pltpu.VMEM((SCHUNK,), jnp.int32),
                       pltpu.VMEM((SCHUNK, D), jnp.float32),
                       pltpu.VMEM((SCHUNK, D), jnp.float32),
                       pltpu.VMEM((SCHUNK,), jnp.float32),
                       pltpu.VMEM_SHARED((ACC_ROWS, D), jnp.float32),
                       pltpu.VMEM_SHARED((ACC_ROWS,), jnp.float32),
                       pltpu.SemaphoreType.DMA,
                       pltpu.SemaphoreType.DMA,
                       pltpu.SemaphoreType.DMA,
                       pltpu.SemaphoreType.DMA],
    )
    return f(bonds_new, idx1)


def _softplus(x):
    return jnp.maximum(x, 0.0) + jnp.log(1.0 + jnp.exp(-jnp.abs(x)))


EBLK = 1600
EGRID = EROWS // EBLK


def _edge_body(a1p, a2p, bdp, st, W1a, W1b, W1c, W1d, b1, b2p, b3p,
               W2bd, W3bd, out, acc):
    i = pl.program_id(0)
    c0 = jnp.dot(st[...], W1d[...], preferred_element_type=jnp.float32) \
        + b1[...]
    c0p = jnp.concatenate([c0, c0, c0, c0], axis=1)
    h = (jnp.dot(a1p[...], W1a[...], preferred_element_type=jnp.float32)
         + jnp.dot(a2p[...], W1b[...], preferred_element_type=jnp.float32)
         + jnp.dot(bdp[...], W1c[...], preferred_element_type=jnp.float32)
         + c0p)
    h = _softplus(h)
    h = _softplus(jnp.dot(h, W2bd[...], preferred_element_type=jnp.float32)
                  + b2p[...])
    h = _softplus(jnp.dot(h, W3bd[...], preferred_element_type=jnp.float32)
                  + b3p[...])
    out[...] = h

    @pl.when(i == 0)
    def _():
        acc[...] = jnp.zeros_like(acc)

    acc[...] += jnp.sum(h.reshape(8, EBLK // 8, 128), axis=1)


def _edge_mlp(a1p, a2p, bdp, state, W1a, W1b, W1c, W1d, b1, b2p, b3p,
              W2bd, W3bd):
    full = lambda shape: pl.BlockSpec(shape, lambda i: (0, 0))
    return pl.pallas_call(
        _edge_body,
        grid=(EGRID,),
        in_specs=[
            pl.BlockSpec((EBLK, 128), lambda i: (i, 0)),
            pl.BlockSpec((EBLK, 128), lambda i: (i, 0)),
            pl.BlockSpec((EBLK, 128), lambda i: (i, 0)),
            full((1, D)),
            full((128, 256)), full((128, 256)), full((128, 256)),
            full((32, 64)), full((1, 64)), full((1, 256)), full((1, 128)),
            full((256, 256)), full((256, 128)),
        ],
        out_specs=[
            pl.BlockSpec((EBLK, 128), lambda i: (i, 0)),
            pl.BlockSpec((8, 128), lambda i: (0, 0)),
        ],
        out_shape=[jax.ShapeDtypeStruct((EROWS, 128), jnp.float32),
                   jax.ShapeDtypeStruct((8, 128), jnp.float32)],
    )(a1p, a2p, bdp, state, W1a, W1b, W1c, W1d, b1, b2p, b3p, W2bd, W3bd)


ABLK = 1000
AGRID = NA // ABLK


def _atom_body(seg, cnt, at, st, W1, b1, W2, b2, W3, b3, out, acc):
    i = pl.program_id(0)
    b2a = seg[...] / cnt[...]
    x = jnp.concatenate([b2a, at[...]], axis=1)
    c0 = jnp.dot(st[...], W1[64:96, :],
                 preferred_element_type=jnp.float32) + b1[...]
    h = jnp.dot(x, W1[0:64, :], preferred_element_type=jnp.float32) + c0
    h = _softplus(h)
    h = _softplus(jnp.dot(h, W2[...], preferred_element_type=jnp.float32)
                  + b2[...])
    h = _softplus(jnp.dot(h, W3[...], preferred_element_type=jnp.float32)
                  + b3[...])
    out[...] = h

    @pl.when(i == 0)
    def _():
        acc[...] = jnp.zeros_like(acc)

    acc[...] += jnp.sum(h.reshape(8, ABLK // 8, D), axis=1)


def _atom_mlp(seg, cnt, atoms, state, W1, b1, W2, b2, W3, b3):
    full = lambda shape: pl.BlockSpec(shape, lambda i: (0, 0))
    return pl.pallas_call(
        _atom_body,
        grid=(AGRID,),
        in_specs=[
            pl.BlockSpec((ABLK, D), lambda i: (i, 0)),
            pl.BlockSpec((ABLK, 1), lambda i: (i, 0)),
            pl.BlockSpec((ABLK, D), lambda i: (i, 0)),
            full((1, D)),
            full((96, 64)), full((1, 64)),
            full((64, 64)), full((1, 64)),
            full((64, 32)), full((1, 32)),
        ],
        out_specs=[
            pl.BlockSpec((ABLK, D), lambda i: (i, 0)),
            pl.BlockSpec((8, D), lambda i: (0, 0)),
        ],
        out_shape=[jax.ShapeDtypeStruct((NA, D), jnp.float32),
                   jax.ShapeDtypeStruct((8, D), jnp.float32)],
    )(seg, cnt, atoms, state, W1, b1, W2, b2, W3, b3)


def _state_body(bacc, aacc, st, W1, b1, W2, b2, W3, b3, out):
    bp = bacc[...]
    bsum = (bp[:, 0:32] + bp[:, 32:64] + bp[:, 64:96] + bp[:, 96:128])
    b2s = jnp.sum(bsum, axis=0, keepdims=True) / NB
    a2s = jnp.sum(aacc[...], axis=0, keepdims=True) / NA
    c0 = jnp.dot(st[...], W1[64:96, :],
                 preferred_element_type=jnp.float32) + b1[...]
    h = (jnp.dot(b2s, W1[0:32, :], preferred_element_type=jnp.float32)
         + jnp.dot(a2s, W1[32:64, :], preferred_element_type=jnp.float32)
         + c0)
    h = _softplus(h)
    h = _softplus(jnp.dot(h, W2[...], preferred_element_type=jnp.float32)
                  + b2[...])
    h = _softplus(jnp.dot(h, W3[...], preferred_element_type=jnp.float32)
                  + b3[...])
    out[...] = h


def _state_mlp(bacc, aacc, state, W1, b1, W2, b2, W3, b3):
    return pl.pallas_call(
        _state_body,
        out_shape=jax.ShapeDtypeStruct((1, D), jnp.float32),
    )(bacc, aacc, state, W1, b1, W2, b2, W3, b3)


def kernel(bonds, bond_atom_1, bond_atom_2, atoms, state,
           e_W1, e_b1, e_W2, e_b2, e_W3, e_b3,
           v_W1, v_b1, v_W2, v_b2, v_W3, v_b3,
           u_W1, u_b1, u_W2, u_b2, u_W3, u_b3):
    a1, a2 = _sc_gather(atoms, bond_atom_1, bond_atom_2)
    eye4 = jnp.eye(4, dtype=jnp.float32)
    bnp, bacc = _edge_mlp(
        a1.reshape(EROWS, 128), a2.reshape(EROWS, 128),
        bonds.reshape(EROWS, 128), state,
        jnp.kron(eye4, e_W1[0:32, :]),
        jnp.kron(eye4, e_W1[32:64, :]),
        jnp.kron(eye4, e_W1[64:96, :]),
        e_W1[96:128, :], e_b1.reshape(1, 64),
        jnp.tile(e_b2, 4).reshape(1, 256), jnp.tile(e_b3, 4).reshape(1, 128),
        jnp.kron(eye4, e_W2), jnp.kron(eye4, e_W3))
    bonds_new = bnp.reshape(NB, D)
    seg, cnt = _sc_scatter(bonds_new, bond_atom_1)
    atoms_new, aacc = _atom_mlp(
        seg, cnt.reshape(NA, 1), atoms, state,
        v_W1, v_b1.reshape(1, 64), v_W2, v_b2.reshape(1, 64),
        v_W3, v_b3.reshape(1, 32))
    state_new = _state_mlp(
        bacc, aacc, state,
        u_W1, u_b1.reshape(1, 64), u_W2, u_b2.reshape(1, 64),
        u_W3, u_b3.reshape(1, 32))
    return (bonds_new, atoms_new, state_new)
